# baseline (speedup 1.0000x reference)
import functools

import jax
import jax.numpy as jnp
from jax.experimental import pallas as pl
from jax.experimental.pallas import tpu as pltpu

NBUF = 8


def _mcopy_body(x_hbm, o_hbm, xbuf, in_sems, out_sems, *, n_img):
    def dma_in(slot, img, pri):
        pltpu.make_async_copy(x_hbm.at[img], xbuf.at[slot],
                              in_sems.at[slot]).start(priority=pri)

    def wait_in(slot):
        pltpu.make_async_copy(xbuf.at[slot], xbuf.at[slot],
                              in_sems.at[slot]).wait()

    def dma_out(slot, img, pri):
        pltpu.make_async_copy(xbuf.at[slot], o_hbm.at[img],
                              out_sems.at[slot]).start(priority=pri)

    def wait_out(slot):
        pltpu.make_async_copy(xbuf.at[slot], xbuf.at[slot],
                              out_sems.at[slot]).wait()

    for k in range(NBUF):
        dma_in(k, k, k % 2)

    def round_body(r, _):
        for k in range(NBUF):
            img = r * NBUF + k
            wait_in(k)
            dma_out(k, img, k % 2)

            @pl.when(img + NBUF < n_img)
            def _():
                wait_out(k)
                dma_in(k, img + NBUF, k % 2)

        return ()

    jax.lax.fori_loop(0, n_img // NBUF, round_body, ())
    for k in range(NBUF):
        wait_out(k)


@jax.jit
def _mcopy_run(x):
    B, C, HW = x.shape
    return pl.pallas_call(
        functools.partial(_mcopy_body, n_img=B),
        out_shape=jax.ShapeDtypeStruct((B, C, HW), x.dtype),
        grid=(1,),
        in_specs=[pl.BlockSpec(memory_space=pl.ANY)],
        out_specs=pl.BlockSpec(memory_space=pl.ANY),
        scratch_shapes=[
            pltpu.VMEM((NBUF, C, HW), jnp.float32),
            pltpu.SemaphoreType.DMA((NBUF,)),
            pltpu.SemaphoreType.DMA((NBUF,)),
        ],
        compiler_params=pltpu.CompilerParams(
            dimension_semantics=("arbitrary",),
            vmem_limit_bytes=40 << 20,
        ),
    )(x)


def kernel(x, w1, b1, w2, b2):
    B, C, H, W = x.shape
    xf = x.reshape(B, C, H * W)
    return _mcopy_run(xf).reshape(B, C, H, W)


# speedup vs baseline: 1.0011x; 1.0011x over previous
import functools

import jax
import jax.numpy as jnp
from jax.experimental import pallas as pl
from jax.experimental.pallas import tpu as pltpu


def _se_body(x_ref, w1_ref, b1_ref, w2_ref, b2_ref, o_ref):
    x = x_ref[...]
    s = jnp.sum(x, axis=-1)
    z = jnp.dot(s, w1_ref[...], preferred_element_type=jnp.float32)
    z = jnp.maximum(z + b1_ref[...], 0.0)
    a = jnp.dot(z, w2_ref[...], preferred_element_type=jnp.float32)
    g = jax.nn.sigmoid(a + b2_ref[...])
    o_ref[...] = x * g[:, :, None]


@functools.partial(jax.jit, static_argnames=("bblk",))
def _se_run(x, w1s, b1r, w2, b2r, *, bblk):
    B, C, HW = x.shape
    Cs = w1s.shape[1]
    grid = B // bblk
    block_bytes = bblk * C * HW * 4
    vmem_limit = 4 * block_bytes + 4 * (C * Cs + Cs * C) + (6 << 20)
    return pl.pallas_call(
        _se_body,
        out_shape=jax.ShapeDtypeStruct((B, C, HW), x.dtype),
        grid=(grid,),
        in_specs=[
            pl.BlockSpec((bblk, C, HW), lambda b: (b, 0, 0)),
            pl.BlockSpec((C, Cs), lambda b: (0, 0)),
            pl.BlockSpec((1, Cs), lambda b: (0, 0)),
            pl.BlockSpec((Cs, C), lambda b: (0, 0)),
            pl.BlockSpec((1, C), lambda b: (0, 0)),
        ],
        out_specs=pl.BlockSpec((bblk, C, HW), lambda b: (b, 0, 0)),
        compiler_params=pltpu.CompilerParams(
            dimension_semantics=("arbitrary",),
            vmem_limit_bytes=int(min(vmem_limit, 100 << 20)),
        ),
        cost_estimate=pl.CostEstimate(
            flops=4 * B * C * Cs + 2 * B * C * HW,
            transcendentals=B * C,
            bytes_accessed=2 * B * C * HW * 4,
        ),
    )(x, w1s, b1r, w2, b2r)


def kernel(x, w1, b1, w2, b2):
    B, C, H, W = x.shape
    HW = H * W
    Cs = w1.shape[1]
    xf = x.reshape(B, C, HW)
    w1s = (w1 / jnp.float32(HW)).astype(jnp.float32)
    out = _se_run(xf, w1s, b1.reshape(1, Cs), w2, b2.reshape(1, C), bblk=4)
    return out.reshape(B, C, H, W)
